# baseline (speedup 1.0000x reference)
import functools
import math

import jax
import jax.numpy as jnp
from jax import lax
from jax.experimental import pallas as pl
from jax.experimental.pallas import tpu as pltpu
from jax.experimental.pallas import tpu_sc as plsc

EMB_D = 32
NUM_CORES = 2
NUM_SUBCORES = 16
NUM_WORKERS = NUM_CORES * NUM_SUBCORES

CHUNK = 800
NBUF = 4


def _body(table_hbm, idx_hbm, out_hbm,
          idx0, idx1, idx2, idx3, rows0, rows1, rows2, rows3,
          gsem0, gsem1, gsem2, gsem3,
          ssem0, ssem1, ssem2, ssem3,
          isem0, isem1, isem2, isem3):
    wid = lax.axis_index("s") * NUM_CORES + lax.axis_index("c")
    b_total = idx_hbm.shape[0]
    b_per_w = b_total // NUM_WORKERS
    n_chunks = b_per_w // CHUNK
    base = wid * b_per_w

    idxs = (idx0, idx1, idx2, idx3)
    rows = (rows0, rows1, rows2, rows3)
    gsems = (gsem0, gsem1, gsem2, gsem3)
    ssems = (ssem0, ssem1, ssem2, ssem3)
    isems = (isem0, isem1, isem2, isem3)

    for g in range(2):
        off = base + g * CHUNK
        pltpu.sync_copy(idx_hbm.at[pl.ds(off, CHUNK)], idxs[g])
        pltpu.async_copy(table_hbm.at[idxs[g]], rows[g], gsems[g])

    def outer(i, _):
        for s in range(NBUF):
            g = i * NBUF + s
            off = base + g * CHUNK
            nxt = g + 2
            sn = (s + 2) % NBUF
            sp = (s - 2) % NBUF

            pltpu.make_async_copy(
                table_hbm.at[idxs[s]], rows[s], gsems[s]).wait()

            @pl.when(nxt < n_chunks)
            def _():
                noff = base + nxt * CHUNK
                pltpu.async_copy(
                    idx_hbm.at[pl.ds(noff, CHUNK)], idxs[sn], isems[sn])

            pltpu.async_copy(rows[s], out_hbm.at[pl.ds(off, CHUNK)], ssems[s])

            @pl.when(g >= 2)
            def _():
                poff = base + (g - 2) * CHUNK
                pltpu.make_async_copy(
                    rows[sp], out_hbm.at[pl.ds(poff, CHUNK)], ssems[sp]).wait()

            @pl.when(nxt < n_chunks)
            def _():
                noff = base + nxt * CHUNK
                pltpu.make_async_copy(
                    idx_hbm.at[pl.ds(noff, CHUNK)], idxs[sn], isems[sn]).wait()
                pltpu.async_copy(table_hbm.at[idxs[sn]], rows[sn], gsems[sn])

        return 0

    lax.fori_loop(0, n_chunks // NBUF, outer, 0)

    for g_last in (n_chunks - 2, n_chunks - 1):
        s = g_last % NBUF
        off = base + g_last * CHUNK
        pltpu.make_async_copy(
            rows[s], out_hbm.at[pl.ds(off, CHUNK)], ssems[s]).wait()


def _gather_rows(table_scaled, idx_flat):
    b_total = idx_flat.shape[0]
    mesh = plsc.VectorSubcoreMesh(core_axis_name="c", subcore_axis_name="s")
    k = functools.partial(
        pl.kernel,
        mesh=mesh,
        out_type=jax.ShapeDtypeStruct((b_total, EMB_D), jnp.float32),
        compiler_params=pltpu.CompilerParams(use_tc_tiling_on_sc=False),
        scratch_types=(
            [pltpu.VMEM((CHUNK,), jnp.int32) for _ in range(NBUF)]
            + [pltpu.VMEM((CHUNK, EMB_D), jnp.float32) for _ in range(NBUF)]
            + [pltpu.SemaphoreType.DMA for _ in range(3 * NBUF)]
        ),
    )(_body)
    return k(table_scaled, idx_flat)


def kernel(tokens, table):
    b_total = tokens.size
    idx = tokens.reshape(b_total).astype(jnp.int32)
    table_scaled = table * jnp.float32(math.sqrt(EMB_D))
    out = _gather_rows(table_scaled, idx)
    return out.reshape(*tokens.shape, EMB_D)


# speedup vs baseline: 1.2153x; 1.2153x over previous
import functools
import math

import jax
import jax.numpy as jnp
from jax import lax
from jax.experimental import pallas as pl
from jax.experimental.pallas import tpu as pltpu
from jax.experimental.pallas import tpu_sc as plsc

EMB_D = 32
LANES = 16
NUM_CORES = 2
NUM_SUBCORES = 16
NUM_WORKERS = NUM_CORES * NUM_SUBCORES

B_DIM = 200
A_DIM = 4096
AB_PER = 4
ROWS = AB_PER * 128
UNITS = B_DIM * (A_DIM // 128) // AB_PER
UNITS_PER_W = UNITS // NUM_WORKERS
NBUF = 2
TRANS_WORDS = 4 * AB_PER * 1024
CH_WORDS = AB_PER * 1024


def _transpose_scale(rows_v, trans_v, scale, pb0):
    def row_step(r2, _):
        for half in range(2):
            r = r2 * 2 + half
            off = (r >> 7) * 1024 + (r & 127)
            idx0 = pb0 + off
            v0 = rows_v[r, pl.ds(0, LANES)] * scale
            plsc.store_scatter(trans_v, [idx0], v0)
            v1 = rows_v[r, pl.ds(LANES, LANES)] * scale
            plsc.store_scatter(trans_v, [idx0 + 8192], v1)
        return 0

    lax.fori_loop(0, ROWS // 2, row_step, 0)


def _body(table_hbm, idx_hbm, out_hbm,
          idx0, idx1, rows0, rows1, trans0, trans1,
          gsem0, gsem1, ssem0, ssem1, isem0, isem1):
    wid = lax.axis_index("s") * NUM_CORES + lax.axis_index("c")
    u0 = wid * UNITS_PER_W
    scale = jnp.float32(math.sqrt(EMB_D))
    c16 = lax.iota(jnp.int32, LANES)
    pb0 = (c16 >> 3) * 4096 + (c16 & 7) * 128

    slots = (
        (idx0, rows0, trans0, gsem0, ssem0, isem0),
        (idx1, rows1, trans1, gsem1, ssem1, isem1),
    )

    def out_word(u):
        b = u // 8
        q = u % 8
        return (b * 128 + q * AB_PER) * 1024

    for s in range(NBUF):
        idx_v, rows_v, _, gsem, _, _ = slots[s]
        pltpu.sync_copy(idx_hbm.at[pl.ds((u0 + s) * ROWS, ROWS)], idx_v)
        pltpu.async_copy(table_hbm.at[idx_v], rows_v, gsem)

    def outer(i, _):
        for s in range(NBUF):
            idx_v, rows_v, trans_v, gsem, ssem, isem = slots[s]
            u = u0 + i * NBUF + s
            nxt = u + NBUF
            last = u0 + UNITS_PER_W

            pltpu.make_async_copy(table_hbm.at[idx_v], rows_v, gsem).wait()

            @pl.when(nxt < last)
            def _():
                pltpu.async_copy(
                    idx_hbm.at[pl.ds(nxt * ROWS, ROWS)], idx_v, isem)

            @pl.when(u - NBUF >= u0)
            def _():
                pw = out_word(u - NBUF)
                for ch in range(4):
                    pltpu.make_async_copy(
                        trans_v.at[pl.ds(ch * CH_WORDS, CH_WORDS)],
                        out_hbm.at[pl.ds(pw + ch * 32 * 1024, CH_WORDS)],
                        ssem).wait()

            _transpose_scale(rows_v, trans_v, scale, pb0)

            w0 = out_word(u)
            for ch in range(4):
                pltpu.async_copy(
                    trans_v.at[pl.ds(ch * CH_WORDS, CH_WORDS)],
                    out_hbm.at[pl.ds(w0 + ch * 32 * 1024, CH_WORDS)],
                    ssem)

            @pl.when(nxt < last)
            def _():
                pltpu.make_async_copy(
                    idx_hbm.at[pl.ds(nxt * ROWS, ROWS)], idx_v, isem).wait()
                pltpu.async_copy(table_hbm.at[idx_v], rows_v, gsem)

        return 0

    lax.fori_loop(0, UNITS_PER_W // NBUF, outer, 0)

    for s in range(NBUF):
        _, _, trans_v, _, ssem, _ = slots[s]
        pw = out_word(u0 + UNITS_PER_W - NBUF + s)
        for ch in range(4):
            pltpu.make_async_copy(
                trans_v.at[pl.ds(ch * CH_WORDS, CH_WORDS)],
                out_hbm.at[pl.ds(pw + ch * 32 * 1024, CH_WORDS)],
                ssem).wait()


def _gather_transposed(table, idx_flat):
    mesh = plsc.VectorSubcoreMesh(core_axis_name="c", subcore_axis_name="s")
    k = functools.partial(
        pl.kernel,
        mesh=mesh,
        out_type=jax.ShapeDtypeStruct((A_DIM * B_DIM * EMB_D,), jnp.float32),
        compiler_params=pltpu.CompilerParams(
            use_tc_tiling_on_sc=False, needs_layout_passes=False,
            disable_bounds_checks=True),
        scratch_types=[
            pltpu.VMEM((ROWS,), jnp.int32),
            pltpu.VMEM((ROWS,), jnp.int32),
            pltpu.VMEM((ROWS, EMB_D), jnp.float32),
            pltpu.VMEM((ROWS, EMB_D), jnp.float32),
            pltpu.VMEM((TRANS_WORDS,), jnp.float32),
            pltpu.VMEM((TRANS_WORDS,), jnp.float32),
            pltpu.SemaphoreType.DMA,
            pltpu.SemaphoreType.DMA,
            pltpu.SemaphoreType.DMA,
            pltpu.SemaphoreType.DMA,
            pltpu.SemaphoreType.DMA,
            pltpu.SemaphoreType.DMA,
        ],
    )(_body)
    return k(table, idx_flat)


def kernel(tokens, table):
    idx_flat = tokens.T.reshape(A_DIM * B_DIM).astype(jnp.int32)
    flat = _gather_transposed(table, idx_flat)
    out5 = flat.reshape(B_DIM, 4, 32, 8, 128)
    return out5.transpose(2, 4, 0, 1, 3).reshape(A_DIM, B_DIM, EMB_D)
